# Initial kernel scaffold; baseline (speedup 1.0000x reference)
import jax
import jax.numpy as jnp
from jax.experimental import pallas as pl


def kernel(z, gt, codebook):
    raise NotImplementedError("write your pallas kernel here")


# speedup vs baseline: 10.1115x; 10.1115x over previous
import functools

import jax
import jax.numpy as jnp
from jax import lax
from jax.experimental import pallas as pl
from jax.experimental.pallas import tpu as pltpu
from jax.experimental.pallas import tpu_sc as plsc

N_CODES = 8192
DIM = 64
N_TOK = 16384
TB = 256
CK = 4096
N_CHUNK = N_CODES // CK


def _round_bf16(v):
    u = lax.bitcast_convert_type(v, jnp.uint32)
    r = (u + jnp.uint32(0x7FFF)) + ((u >> 16) & jnp.uint32(1))
    return lax.bitcast_convert_type(r & jnp.uint32(0xFFFF0000), jnp.float32)


def _argmin_body(x2_ref, x_ref, c2_ref, cbt_ref, idx_ref):
    x = x_ref[...]
    x2 = x2_ref[...]

    col = lax.broadcasted_iota(jnp.int32, (TB, CK), 1).astype(jnp.float32)

    def step(i, carry):
        run_min, run_loc, run_chunk = carry
        start = pl.multiple_of(i * CK, CK)
        cbt = cbt_ref[:, pl.ds(start, CK)]
        c2 = c2_ref[:, pl.ds(start, CK)]
        zc = lax.dot_general(x.astype(jnp.bfloat16), cbt.astype(jnp.bfloat16),
                             (((1,), (0,)), ((), ())),
                             preferred_element_type=jnp.float32)
        d = (x2 + c2) + zc
        m = jnp.min(d, axis=1, keepdims=True)
        ci = jnp.min(jnp.where(d == m, col, 65536.0), axis=1, keepdims=True)
        better = m < run_min
        chunk_f = lax.convert_element_type(i, jnp.float32)
        new_min = _round_bf16(jnp.where(better, m, run_min))
        return (new_min,
                jnp.where(better, ci, run_loc),
                jnp.where(better, chunk_f, run_chunk))

    init = (jnp.full((TB, 1), jnp.inf, jnp.float32),
            jnp.zeros((TB, 1), jnp.float32),
            jnp.zeros((TB, 1), jnp.float32))
    _, run_loc, run_chunk = lax.fori_loop(0, N_CHUNK, step, init)
    idx_ref[...] = (run_chunk * float(CK) + run_loc).astype(jnp.int32)


def _argmin_call(x2, x, c2, cbt):
    return pl.pallas_call(
        _argmin_body,
        grid=(N_TOK // TB,),
        in_specs=[
            pl.BlockSpec((TB, 1), lambda t: (t, 0)),
            pl.BlockSpec((TB, DIM), lambda t: (t, 0)),
            pl.BlockSpec((1, N_CODES), lambda t: (0, 0)),
            pl.BlockSpec((DIM, N_CODES), lambda t: (0, 0)),
        ],
        out_specs=pl.BlockSpec((TB, 1), lambda t: (t, 0)),
        out_shape=jax.ShapeDtypeStruct((N_TOK, 1), jnp.int32),
    )(x2, x, c2, cbt)


def _make_gather():
    info = plsc.get_sparse_core_info()
    nc, ns = info.num_cores, info.num_subcores
    nw = nc * ns
    rows_per_w = N_TOK // nw
    n_sub = rows_per_w // 128

    mesh = plsc.VectorSubcoreMesh(core_axis_name="c", subcore_axis_name="s")

    @functools.partial(
        pl.kernel, mesh=mesh,
        out_type=jax.ShapeDtypeStruct((N_TOK, 128), jnp.float32),
        scratch_types=[
            pltpu.VMEM((n_sub, 128), jnp.int32),
            pltpu.VMEM((rows_per_w, 128), jnp.float32),
            pltpu.SemaphoreType.DMA,
        ],
    )
    def gather_rows(idx_hbm, table_hbm, out_hbm, idx_v, rows_v, sem):
        wid = lax.axis_index("s") * nc + lax.axis_index("c")
        pltpu.sync_copy(idx_hbm.at[pl.ds(wid * n_sub, n_sub)], idx_v)
        copies = [
            pltpu.async_copy(table_hbm.at[idx_v.at[j]],
                             rows_v.at[pl.ds(j * 128, 128)], sem)
            for j in range(n_sub)
        ]
        for c in copies:
            c.wait()
        pltpu.sync_copy(rows_v, out_hbm.at[pl.ds(wid * rows_per_w, rows_per_w)])

    return gather_rows


_gather_rows = None


def _cosine_body(a_ref, b_ref, o_ref):
    a = a_ref[0][:, :, :DIM]
    b = b_ref[0][:, :, :DIM]
    num = jnp.sum(a * b, axis=0)
    an = jnp.sqrt(jnp.sum(a * a, axis=0))
    bn = jnp.sqrt(jnp.sum(b * b, axis=0))
    cos = num / (jnp.maximum(an, 1e-8) * jnp.maximum(bn, 1e-8))
    o_ref[0] = jnp.round(cos * 1e4) / 1e4


def _cosine_call(a, b):
    return pl.pallas_call(
        _cosine_body,
        grid=(8,),
        in_specs=[
            pl.BlockSpec((1, 32, 32, 128), lambda i: (i, 0, 0, 0)),
            pl.BlockSpec((1, 32, 32, 128), lambda i: (i, 0, 0, 0)),
        ],
        out_specs=pl.BlockSpec((1, 32, DIM), lambda i: (i, 0, 0)),
        out_shape=jax.ShapeDtypeStruct((8, 32, DIM), jnp.float32),
    )(a, b)


def kernel(z, gt, codebook):
    global _gather_rows
    if _gather_rows is None:
        _gather_rows = _make_gather()

    zp = jnp.transpose(z, (0, 2, 3, 1)).reshape(-1, DIM)
    gtp = jnp.transpose(gt, (0, 2, 3, 1)).reshape(-1, DIM)
    x = jnp.concatenate([zp, gtp], axis=0)
    x2 = jnp.sum(x ** 2, axis=1, keepdims=True)
    c2 = jnp.sum(codebook ** 2, axis=1)[None, :]
    cbt = codebook.T

    idx = _argmin_call(x2, -2.0 * x, c2, cbt)

    cb_pad = jnp.pad(codebook, ((0, 0), (0, 128 - DIM)))
    rows = _gather_rows(idx.reshape(-1, 128), cb_pad)
    half = N_TOK // 2
    zq = rows[:half].reshape(8, 32, 32, 128)
    zq_gt = rows[half:].reshape(8, 32, 32, 128)

    cos = _cosine_call(zq_gt, zq)
    return (cos, idx[half:], idx[:half])
